# baseline (speedup 1.0000x reference)
import functools

import jax
import jax.numpy as jnp
from jax import lax
from jax.experimental import pallas as pl
from jax.experimental.pallas import tpu as pltpu
from jax.experimental.pallas import tpu_sc as plsc

_N = 100000
_V = 119
_D = 128
_NW = 32
_CHUNK = 128
_NCH = 26
_NBUF = 7
_LOOKAHEAD = 3
_PW = _CHUNK * _NCH
_LAST = _N - _PW


def _make_kernel():
    mesh = plsc.VectorSubcoreMesh(core_axis_name="c", subcore_axis_name="s")

    @functools.partial(
        pl.kernel,
        mesh=mesh,
        out_type=jax.ShapeDtypeStruct((_N, _D), jnp.float32),
        scratch_types=[
            pltpu.VMEM_SHARED((_V, _D), jnp.float32),
            pltpu.VMEM((_PW,), jnp.int32),
            pltpu.VMEM((_NBUF, _CHUNK, _D), jnp.float32),
        ]
        + [pltpu.SemaphoreType.DMA] * (2 * _NBUF),
    )
    def emb_kernel(z_hbm, table_hbm, out_hbm, table_sh, idx_v, rows_v, *sems):
        sg = sems[:_NBUF]
        sw = sems[_NBUF:]
        wid = lax.axis_index("s") * 2 + lax.axis_index("c")
        base = ((wid * _LAST) // (_NW - 1)) // 8 * 8

        @pl.when(lax.axis_index("s") == 0)
        def _():
            pltpu.sync_copy(table_hbm, table_sh)

        pltpu.sync_copy(z_hbm.at[pl.ds(base, _PW)], idx_v)
        plsc.subcore_barrier()

        def gather(c, b):
            return pltpu.make_async_copy(
                table_sh.at[idx_v.at[pl.ds(c * _CHUNK, _CHUNK)]],
                rows_v.at[b],
                sg[b],
            )

        def write(c, b):
            return pltpu.make_async_copy(
                rows_v.at[b],
                out_hbm.at[pl.ds(base + c * _CHUNK, _CHUNK)],
                sw[b],
            )

        def chunk_step(c, b):
            gather(c, b).wait()
            write(c, b).start()

            @pl.when(c >= _LOOKAHEAD)
            def _():
                write(c - _LOOKAHEAD, (b - _LOOKAHEAD) % _NBUF).wait()

            @pl.when(c + _NBUF - _LOOKAHEAD < _NCH)
            def _():
                nxt = c + _NBUF - _LOOKAHEAD
                gather(nxt, (b + _NBUF - _LOOKAHEAD) % _NBUF).start()

        for c in range(_NBUF - _LOOKAHEAD):
            gather(c, c).start()

        def body(i, carry):
            for b in range(_NBUF):
                c = _NBUF * i + b

                @pl.when(c < _NCH)
                def _(c=c, b=b):
                    chunk_step(c, b)

            return carry

        lax.fori_loop(0, (_NCH + _NBUF - 1) // _NBUF, body, 0)
        for c in range(_NCH - _LOOKAHEAD, _NCH):
            write(c, c % _NBUF).wait()

    return emb_kernel


_emb = _make_kernel()


def kernel(Z, embeddings):
    return _emb(Z.astype(jnp.int32), embeddings)


# speedup vs baseline: 1.0423x; 1.0423x over previous
import functools

import jax
import jax.numpy as jnp
from jax import lax
from jax.experimental import pallas as pl
from jax.experimental.pallas import tpu as pltpu
from jax.experimental.pallas import tpu_sc as plsc

_N = 100000
_V = 119
_D = 128
_NW = 32
_CHUNK = 128
_FULL = 24
_NCH = _FULL + 1
_NBUF = 6
_LOOKAHEAD = 3
_STAGE = _FULL * _CHUNK


def _make_kernel():
    mesh = plsc.VectorSubcoreMesh(core_axis_name="c", subcore_axis_name="s")

    @functools.partial(
        pl.kernel,
        mesh=mesh,
        out_type=jax.ShapeDtypeStruct((_N, _D), jnp.float32),
        scratch_types=[
            pltpu.VMEM_SHARED((_V, _D), jnp.float32),
            pltpu.VMEM((_STAGE + _CHUNK,), jnp.int32),
            pltpu.VMEM((_NBUF, _CHUNK, _D), jnp.float32),
        ]
        + [pltpu.SemaphoreType.DMA] * (2 * _NBUF + 1),
    )
    def emb_kernel(z_hbm, table_hbm, out_hbm, table_sh, idx_v, rows_v, *sems):
        sg = sems[:_NBUF]
        sw = sems[_NBUF:2 * _NBUF]
        si = sems[2 * _NBUF]
        wid = lax.axis_index("s") * 2 + lax.axis_index("c")
        base = (wid * (_N // _NW)) // 8 * 8
        nxt = jnp.where(wid == _NW - 1, _N, ((wid + 1) * (_N // _NW)) // 8 * 8)
        tail = nxt - _CHUNK

        idx_main = pltpu.make_async_copy(
            z_hbm.at[pl.ds(base, _STAGE)], idx_v.at[pl.ds(0, _STAGE)], si)
        idx_tail = pltpu.make_async_copy(
            z_hbm.at[pl.ds(tail, _CHUNK)], idx_v.at[pl.ds(_STAGE, _CHUNK)], si)
        idx_main.start()
        idx_tail.start()

        @pl.when(lax.axis_index("s") == 0)
        def _():
            pltpu.sync_copy(table_hbm, table_sh)

        idx_main.wait()
        idx_tail.wait()
        plsc.subcore_barrier()

        def out_off(c):
            return jnp.where(c == _FULL, tail, base + c * _CHUNK)

        def gather(c, b):
            return pltpu.make_async_copy(
                table_sh.at[idx_v.at[pl.ds(c * _CHUNK, _CHUNK)]],
                rows_v.at[b],
                sg[b],
            )

        def write(c, b):
            return pltpu.make_async_copy(
                rows_v.at[b],
                out_hbm.at[pl.ds(out_off(c), _CHUNK)],
                sw[b],
            )

        def chunk_step(c, b):
            gather(c, b).wait()
            write(c, b).start()

            @pl.when(c >= _LOOKAHEAD)
            def _():
                write(c - _LOOKAHEAD, (b - _LOOKAHEAD) % _NBUF).wait()

            @pl.when(c + _NBUF - _LOOKAHEAD < _NCH)
            def _():
                gather(c + _NBUF - _LOOKAHEAD,
                       (b + _NBUF - _LOOKAHEAD) % _NBUF).start()

        for c in range(_NBUF - _LOOKAHEAD):
            gather(c, c).start()

        def body(i, carry):
            for b in range(_NBUF):
                c = _NBUF * i + b

                @pl.when(c < _NCH)
                def _(c=c, b=b):
                    chunk_step(c, b)

            return carry

        lax.fori_loop(0, (_NCH + _NBUF - 1) // _NBUF, body, 0)
        for c in range(_NCH - _LOOKAHEAD, _NCH):
            write(c, c % _NBUF).wait()

    return emb_kernel


_emb = _make_kernel()


def kernel(Z, embeddings):
    return _emb(Z.astype(jnp.int32), embeddings)
